# baseline (device time: 11623 ns/iter reference)
import jax
import jax.numpy as jnp
from jax import lax
from jax.experimental import pallas as pl
from jax.experimental.pallas import tpu as pltpu

T = 256
D = 512
V_LOC = 4096
NV = 4
VC = V_LOC // NV
NBUF = 3


def kernel(x, W, labels):
    def body(x_hbm, w_hbm, lab_hbm, out_ref,
             x_vmem, w_vmem, lab_vmem, my_ref, peer_ref,
             x_sem, lab_sem, w_sems, send_sem, recv_sem):
        my_x = lax.axis_index("x")
        my_y = lax.axis_index("y")
        my_z = lax.axis_index("z")

        barrier_sem = pltpu.get_barrier_semaphore()
        pl.semaphore_signal(
            barrier_sem, inc=1,
            device_id=(my_x, 1 - my_y, my_z),
            device_id_type=pltpu.DeviceIdType.MESH,
        )

        def w_copy(v):
            return pltpu.make_async_copy(
                w_hbm.at[:, pl.ds(v * VC, VC)], w_vmem.at[v % NBUF],
                w_sems.at[v % NBUF],
            )

        x_copy = pltpu.make_async_copy(x_hbm, x_vmem, x_sem)
        x_copy.start()
        lab_copy = pltpu.make_async_copy(lab_hbm, lab_vmem, lab_sem)
        lab_copy.start()
        for v in range(NBUF - 1):
            w_copy(v).start()
        lab_copy.wait()

        eye = (lax.broadcasted_iota(jnp.int32, (T, T), 0)
               == lax.broadcasted_iota(jnp.int32, (T, T), 1)).astype(jnp.float32)
        lab_row = lab_vmem[...].astype(jnp.float32).reshape(1, T)
        col0 = jnp.round(lax.dot_general(
            eye, lab_row, (((1,), (1,)), ((), ())),
            preferred_element_type=jnp.float32,
            precision=lax.Precision.HIGHEST,
        )) - (my_y * V_LOC).astype(jnp.float32)
        ids = lax.broadcasted_iota(jnp.int32, (T, VC), 1).astype(jnp.float32)

        x_copy.wait()
        xv = x_vmem[...]
        m_acc = s_acc = g_acc = None

        for v in range(NV):
            w_copy(v).wait()
            if v + NBUF - 1 < NV:
                w_copy(v + NBUF - 1).start()
            logits = jnp.dot(xv, w_vmem[v % NBUF],
                             preferred_element_type=jnp.float32)
            m_c = jnp.max(logits, axis=1, keepdims=True)
            s_c = jnp.sum(jnp.exp(logits - m_c), axis=1, keepdims=True)
            g_c = jnp.sum(jnp.where(ids == col0 - v * VC, logits, 0.0),
                          axis=1, keepdims=True)
            if v == 0:
                m_acc, s_acc, g_acc = m_c, s_c, g_c
            else:
                m_new = jnp.maximum(m_acc, m_c)
                s_acc = s_acc * jnp.exp(m_acc - m_new) + s_c * jnp.exp(m_c - m_new)
                m_acc = m_new
                g_acc = g_acc + g_c

        my_ref[:, 0:1] = m_acc
        my_ref[:, 1:2] = s_acc
        my_ref[:, 2:3] = g_acc

        pl.semaphore_wait(barrier_sem, 1)

        rdma = pltpu.make_async_remote_copy(
            src_ref=my_ref,
            dst_ref=peer_ref,
            send_sem=send_sem,
            recv_sem=recv_sem,
            device_id=(my_x, 1 - my_y, my_z),
            device_id_type=pltpu.DeviceIdType.MESH,
        )
        rdma.start()
        rdma.wait()

        m_r = peer_ref[:, 0:1]
        s_r = peer_ref[:, 1:2]
        g_r = peer_ref[:, 2:3]
        m = jnp.maximum(m_acc, m_r)
        s = s_acc * jnp.exp(m_acc - m) + s_r * jnp.exp(m_r - m)
        nll = m + jnp.log(s) - (g_acc + g_r)
        nll_row = lax.dot_general(
            nll, eye, (((0,), (0,)), ((), ())),
            preferred_element_type=jnp.float32,
            precision=lax.Precision.HIGHEST,
        )
        out_ref[...] = nll_row.reshape(T)

    out = pl.pallas_call(
        body,
        out_shape=jax.ShapeDtypeStruct((T,), jnp.float32),
        in_specs=[
            pl.BlockSpec(memory_space=pltpu.MemorySpace.HBM),
            pl.BlockSpec(memory_space=pltpu.MemorySpace.HBM),
            pl.BlockSpec(memory_space=pltpu.MemorySpace.HBM),
        ],
        out_specs=pl.BlockSpec(memory_space=pltpu.MemorySpace.VMEM),
        scratch_shapes=[
            pltpu.VMEM((T, D), jnp.float32),
            pltpu.VMEM((NBUF, D, VC), jnp.float32),
            pltpu.VMEM((T,), jnp.int32),
            pltpu.VMEM((T, 8), jnp.float32),
            pltpu.VMEM((T, 8), jnp.float32),
            pltpu.SemaphoreType.DMA,
            pltpu.SemaphoreType.DMA,
            pltpu.SemaphoreType.DMA((NBUF,)),
            pltpu.SemaphoreType.DMA,
            pltpu.SemaphoreType.DMA,
        ],
        compiler_params=pltpu.CompilerParams(collective_id=0),
    )(
        pltpu.with_memory_space_constraint(x, pltpu.MemorySpace.HBM),
        pltpu.with_memory_space_constraint(W, pltpu.MemorySpace.HBM),
        pltpu.with_memory_space_constraint(labels, pltpu.MemorySpace.HBM),
    )
    return out
